# baseline (speedup 1.0000x reference)
import functools

import jax
import jax.numpy as jnp
from jax import lax
from jax.experimental import pallas as pl
from jax.experimental.pallas import tpu as pltpu
from jax.experimental.pallas import tpu_sc as plsc

S = 1024
BN = 2048


def _k1_body(ix_ref, x_ref, wgT_ref, bg_ref, wfT_ref, bf_ref, whT_ref, bh_ref,
             hy_ref, num_ref, den_ref):
    n = pl.program_id(1)
    nb = pl.num_programs(1)
    xb = x_ref[0]
    g = jnp.dot(xb, wgT_ref[...], preferred_element_type=jnp.float32) + bg_ref[...]
    e = jnp.exp(g)
    f = jnp.dot(xb, wfT_ref[...], preferred_element_type=jnp.float32) + bf_ref[...]
    fe = (f * e).astype(jnp.bfloat16)
    eb = e.astype(jnp.bfloat16)
    ixb = ix_ref[0]
    onehotT = (lax.broadcasted_iota(jnp.int32, (S, BN), 0) == ixb
               ).astype(jnp.bfloat16)
    pden = jnp.dot(onehotT, eb, preferred_element_type=jnp.float32)
    pnum = jnp.dot(onehotT, fe, preferred_element_type=jnp.float32)

    @pl.when(n == 0)
    def _init():
        num_ref[...] = pnum
        den_ref[...] = pden

    @pl.when(n > 0)
    def _acc():
        num_ref[...] += pnum
        den_ref[...] += pden

    @pl.when(n == nb - 1)
    def _final():
        num = num_ref[...]
        den = den_ref[...]
        y = jnp.where(den > 0, num / den, 0.0)
        hy = jnp.dot(y, whT_ref[...], preferred_element_type=jnp.float32) + bh_ref[...]
        hy_ref[...] = hy[None]


def _make_sc_gather(BN_total, D, rows_per_w, chunk):
    nchunk = rows_per_w // chunk
    mesh = plsc.VectorSubcoreMesh(core_axis_name="c", subcore_axis_name="s")

    @functools.partial(
        pl.kernel, mesh=mesh,
        out_type=jax.ShapeDtypeStruct((BN_total, D), jnp.float32),
        scratch_types=[
            pltpu.VMEM((rows_per_w,), jnp.int32),
            pltpu.VMEM((chunk, D), jnp.float32),
            pltpu.VMEM((chunk, D), jnp.float32),
            pltpu.SemaphoreType.DMA,
            pltpu.SemaphoreType.DMA,
        ],
    )
    def gather(table_hbm, idx_hbm, out_hbm, idx_v, rows_a, rows_b, sem_a, sem_b):
        wid = lax.axis_index("s") * 2 + lax.axis_index("c")
        base = wid * rows_per_w
        pltpu.sync_copy(idx_hbm.at[pl.ds(base, rows_per_w)], idx_v)
        bufs = (rows_a, rows_b)
        sems = (sem_a, sem_b)
        pltpu.async_copy(table_hbm.at[idx_v.at[pl.ds(0, chunk)]],
                         bufs[0], sems[0])
        for c in range(nchunk):
            cur = c % 2
            if c + 1 < nchunk:
                pltpu.async_copy(
                    table_hbm.at[idx_v.at[pl.ds((c + 1) * chunk, chunk)]],
                    bufs[1 - cur], sems[1 - cur])
            pltpu.make_async_copy(
                table_hbm.at[idx_v.at[pl.ds(c * chunk, chunk)]],
                bufs[cur], sems[cur]).wait()
            pltpu.sync_copy(bufs[cur], out_hbm.at[pl.ds(base + c * chunk, chunk)])

    return gather


def kernel(x, ix, Wf, bf, Wg, bg, Wh, bh):
    B, N, D = x.shape
    NB = N // BN
    ixi = ix.astype(jnp.int32)
    ix3 = ixi.reshape(NB, 1, BN)
    bg2 = bg.reshape(1, D)
    bf2 = bf.reshape(1, D)
    bh2 = bh.reshape(1, D)
    wgT = Wg.T
    wfT = Wf.T
    whT = Wh.T

    k1 = pl.pallas_call(
        _k1_body,
        grid=(1, NB),
        in_specs=[
            pl.BlockSpec((1, 1, BN), lambda b, n: (n, 0, 0)),
            pl.BlockSpec((1, BN, D), lambda b, n: (b, n, 0)),
            pl.BlockSpec((D, D), lambda b, n: (0, 0)),
            pl.BlockSpec((1, D), lambda b, n: (0, 0)),
            pl.BlockSpec((D, D), lambda b, n: (0, 0)),
            pl.BlockSpec((1, D), lambda b, n: (0, 0)),
            pl.BlockSpec((D, D), lambda b, n: (0, 0)),
            pl.BlockSpec((1, D), lambda b, n: (0, 0)),
        ],
        out_specs=pl.BlockSpec((1, S, D), lambda b, n: (b, 0, 0)),
        out_shape=jax.ShapeDtypeStruct((1, S, D), jnp.float32),
        scratch_shapes=[
            pltpu.VMEM((S, D), jnp.float32),
            pltpu.VMEM((S, D), jnp.float32),
        ],
    )

    rows_per_w = N // 32
    sc_gather = _make_sc_gather(N, D, rows_per_w, 64)
    outs = []
    for b in range(B):
        hy_b = k1(ix3, x[b:b + 1], wgT, bg2, wfT, bf2, whT, bh2)
        outs.append(sc_gather(hy_b.reshape(S, D), ixi))
    return jnp.stack(outs, 0)


# speedup vs baseline: 1.2550x; 1.2550x over previous
import functools

import jax
import jax.numpy as jnp
from jax import lax
from jax.experimental import pallas as pl
from jax.experimental.pallas import tpu as pltpu
from jax.experimental.pallas import tpu_sc as plsc

S = 1024
BN = 4096


def _k1_body(ix_ref, x_ref, wgT_ref, bg_ref, wfT_ref, bf_ref, whT_ref, bh_ref,
             hy_ref, num_ref, den_ref):
    n = pl.program_id(1)
    nb = pl.num_programs(1)
    xb = x_ref[0]
    g = jnp.dot(xb, wgT_ref[...], preferred_element_type=jnp.float32) + bg_ref[...]
    e = jnp.exp(g)
    f = jnp.dot(xb, wfT_ref[...], preferred_element_type=jnp.float32) + bf_ref[...]
    fe = (f * e).astype(jnp.bfloat16)
    eb = e.astype(jnp.bfloat16)
    ixb = ix_ref[0]
    onehotT = (lax.broadcasted_iota(jnp.int32, (S, BN), 0) == ixb
               ).astype(jnp.bfloat16)
    pden = jnp.dot(onehotT, eb, preferred_element_type=jnp.float32)
    pnum = jnp.dot(onehotT, fe, preferred_element_type=jnp.float32)

    @pl.when(n == 0)
    def _init():
        num_ref[...] = pnum
        den_ref[...] = pden

    @pl.when(n > 0)
    def _acc():
        num_ref[...] += pnum
        den_ref[...] += pden

    @pl.when(n == nb - 1)
    def _final():
        num = num_ref[...]
        den = den_ref[...]
        y = jnp.where(den > 0, num / den, 0.0)
        hy = jnp.dot(y, whT_ref[...], preferred_element_type=jnp.float32) + bh_ref[...]
        hy_ref[...] = hy[None]


def _make_sc_gather(BN_total, D, rows_per_w, chunk):
    nchunk = rows_per_w // chunk
    mesh = plsc.VectorSubcoreMesh(core_axis_name="c", subcore_axis_name="s")

    @functools.partial(
        pl.kernel, mesh=mesh,
        out_type=jax.ShapeDtypeStruct((BN_total, D), jnp.float32),
        scratch_types=[
            pltpu.VMEM((rows_per_w,), jnp.int32),
            pltpu.VMEM((chunk, D), jnp.float32),
            pltpu.VMEM((chunk, D), jnp.float32),
            pltpu.SemaphoreType.DMA,
            pltpu.SemaphoreType.DMA,
        ],
    )
    def gather(table_hbm, idx_hbm, out_hbm, idx_v, rows_a, rows_b, sem_a, sem_b):
        wid = lax.axis_index("s") * 2 + lax.axis_index("c")
        base = wid * rows_per_w
        pltpu.sync_copy(idx_hbm.at[pl.ds(base, rows_per_w)], idx_v)
        bufs = (rows_a, rows_b)
        sems = (sem_a, sem_b)
        pltpu.async_copy(table_hbm.at[idx_v.at[pl.ds(0, chunk)]],
                         bufs[0], sems[0])
        for c in range(nchunk):
            cur = c % 2
            if c + 1 < nchunk:
                pltpu.async_copy(
                    table_hbm.at[idx_v.at[pl.ds((c + 1) * chunk, chunk)]],
                    bufs[1 - cur], sems[1 - cur])
            pltpu.make_async_copy(
                table_hbm.at[idx_v.at[pl.ds(c * chunk, chunk)]],
                bufs[cur], sems[cur]).wait()
            pltpu.sync_copy(bufs[cur], out_hbm.at[pl.ds(base + c * chunk, chunk)])

    return gather


def kernel(x, ix, Wf, bf, Wg, bg, Wh, bh):
    B, N, D = x.shape
    NB = N // BN
    ixi = ix.astype(jnp.int32)
    ix3 = ixi.reshape(NB, 1, BN)
    bg2 = bg.reshape(1, D)
    bf2 = bf.reshape(1, D)
    bh2 = bh.reshape(1, D)
    wgT = Wg.T
    wfT = Wf.T
    whT = Wh.T

    hy = pl.pallas_call(
        _k1_body,
        grid=(B, NB),
        in_specs=[
            pl.BlockSpec((1, 1, BN), lambda b, n: (n, 0, 0)),
            pl.BlockSpec((1, BN, D), lambda b, n: (b, n, 0)),
            pl.BlockSpec((D, D), lambda b, n: (0, 0)),
            pl.BlockSpec((1, D), lambda b, n: (0, 0)),
            pl.BlockSpec((D, D), lambda b, n: (0, 0)),
            pl.BlockSpec((1, D), lambda b, n: (0, 0)),
            pl.BlockSpec((D, D), lambda b, n: (0, 0)),
            pl.BlockSpec((1, D), lambda b, n: (0, 0)),
        ],
        out_specs=pl.BlockSpec((1, S, D), lambda b, n: (b, 0, 0)),
        out_shape=jax.ShapeDtypeStruct((B, S, D), jnp.float32),
        scratch_shapes=[
            pltpu.VMEM((S, D), jnp.float32),
            pltpu.VMEM((S, D), jnp.float32),
        ],
    )(ix3, x, wgT, bg2, wfT, bf2, whT, bh2)

    idx_full = (ixi[None, :] + S * jnp.arange(B, dtype=jnp.int32)[:, None]
                ).reshape(B * N)
    rows_per_w = (B * N) // 32
    sc_gather = _make_sc_gather(B * N, D, rows_per_w, 64)
    out = sc_gather(hy.reshape(B * S, D), idx_full)
    return out.reshape(B, N, D)


# speedup vs baseline: 1.2652x; 1.0082x over previous
import functools

import jax
import jax.numpy as jnp
from jax import lax
from jax.experimental import pallas as pl
from jax.experimental.pallas import tpu as pltpu
from jax.experimental.pallas import tpu_sc as plsc

S = 1024
BN = 4096


def _k1_body(ix_ref, x_ref, wgf_ref, bgf_ref, whT_ref, bh_ref,
             hy_ref, acc_ref):
    n = pl.program_id(1)
    nb = pl.num_programs(1)
    D = whT_ref.shape[0]
    xb = x_ref[0]
    gf = jnp.dot(xb, wgf_ref[...], preferred_element_type=jnp.float32) + bgf_ref[...]
    g = gf[:, :D]
    f = gf[:, D:]
    e = jnp.exp(g)
    efe = jnp.concatenate([e, f * e], axis=1).astype(jnp.bfloat16)
    ixb = ix_ref[0]
    onehotT = (lax.broadcasted_iota(jnp.int32, (S, BN), 0) == ixb
               ).astype(jnp.bfloat16)
    part = jnp.dot(onehotT, efe, preferred_element_type=jnp.float32)

    @pl.when(n == 0)
    def _init():
        acc_ref[...] = part

    @pl.when(n > 0)
    def _acc():
        acc_ref[...] += part

    @pl.when(n == nb - 1)
    def _final():
        acc = acc_ref[...]
        y = acc[:, D:] / (acc[:, :D] + 1e-30)
        hy = jnp.dot(y, whT_ref[...], preferred_element_type=jnp.float32) + bh_ref[...]
        hy_ref[...] = hy[None]


def _make_sc_gather(BN_total, D, rows_per_w, chunk):
    nchunk = rows_per_w // chunk
    mesh = plsc.VectorSubcoreMesh(core_axis_name="c", subcore_axis_name="s")

    @functools.partial(
        pl.kernel, mesh=mesh,
        out_type=jax.ShapeDtypeStruct((BN_total, D), jnp.float32),
        scratch_types=[
            pltpu.VMEM((rows_per_w,), jnp.int32),
            pltpu.VMEM((chunk, D), jnp.float32),
            pltpu.VMEM((chunk, D), jnp.float32),
            pltpu.SemaphoreType.DMA,
            pltpu.SemaphoreType.DMA,
        ],
    )
    def gather(table_hbm, idx_hbm, out_hbm, idx_v, rows_a, rows_b, sem_a, sem_b):
        wid = lax.axis_index("s") * 2 + lax.axis_index("c")
        base = wid * rows_per_w
        pltpu.sync_copy(idx_hbm.at[pl.ds(base, rows_per_w)], idx_v)
        bufs = (rows_a, rows_b)
        sems = (sem_a, sem_b)
        pltpu.async_copy(table_hbm.at[idx_v.at[pl.ds(0, chunk)]],
                         bufs[0], sems[0])
        for c in range(nchunk):
            cur = c % 2
            if c + 1 < nchunk:
                pltpu.async_copy(
                    table_hbm.at[idx_v.at[pl.ds((c + 1) * chunk, chunk)]],
                    bufs[1 - cur], sems[1 - cur])
            pltpu.make_async_copy(
                table_hbm.at[idx_v.at[pl.ds(c * chunk, chunk)]],
                bufs[cur], sems[cur]).wait()
            pltpu.sync_copy(bufs[cur], out_hbm.at[pl.ds(base + c * chunk, chunk)])

    return gather


def kernel(x, ix, Wf, bf, Wg, bg, Wh, bh):
    B, N, D = x.shape
    NB = N // BN
    ixi = ix.astype(jnp.int32)
    ix3 = ixi.reshape(NB, 1, BN)
    bh2 = bh.reshape(1, D)
    wgf = jnp.concatenate([Wg.T, Wf.T], axis=1)
    bgf = jnp.concatenate([bg, bf]).reshape(1, 2 * D)
    whT = Wh.T

    hy = pl.pallas_call(
        _k1_body,
        grid=(B, NB),
        in_specs=[
            pl.BlockSpec((1, 1, BN), lambda b, n: (n, 0, 0)),
            pl.BlockSpec((1, BN, D), lambda b, n: (b, n, 0)),
            pl.BlockSpec((D, 2 * D), lambda b, n: (0, 0)),
            pl.BlockSpec((1, 2 * D), lambda b, n: (0, 0)),
            pl.BlockSpec((D, D), lambda b, n: (0, 0)),
            pl.BlockSpec((1, D), lambda b, n: (0, 0)),
        ],
        out_specs=pl.BlockSpec((1, S, D), lambda b, n: (b, 0, 0)),
        out_shape=jax.ShapeDtypeStruct((B, S, D), jnp.float32),
        scratch_shapes=[
            pltpu.VMEM((S, 2 * D), jnp.float32),
        ],
    )(ix3, x, wgf, bgf, whT, bh2)

    idx_full = (ixi[None, :] + S * jnp.arange(B, dtype=jnp.int32)[:, None]
                ).reshape(B * N)
    rows_per_w = (B * N) // 32
    sc_gather = _make_sc_gather(B * N, D, rows_per_w, 64)
    out = sc_gather(hy.reshape(B * S, D), idx_full)
    return out.reshape(B, N, D)
